# Initial kernel scaffold; baseline (speedup 1.0000x reference)
import jax
import jax.numpy as jnp
from jax.experimental import pallas as pl


def kernel(h, edge_index, rbf, W1, b1, W2, b2, U1, c1, U2, c2):
    raise NotImplementedError("write your pallas kernel here")


# speedup vs baseline: 2.1677x; 2.1677x over previous
import functools

import jax
import jax.numpy as jnp
from jax import lax
from jax.experimental import pallas as pl
from jax.experimental.pallas import tpu as pltpu
from jax.experimental.pallas import tpu_sc as plsc

N_NODES = 10000
N_EDGES = 160000
HIDDEN = 256
HALF = 128
N_RBF = 16

NC = 2
NS = 16
CHUNK = 80
EDGES_PER_SUB = N_EDGES // NS
NCHUNK = EDGES_PER_SUB // CHUNK
N_PAD = 10240
ROWS_PER_SUB = N_PAD // NS


def _silu(x):
    return x * jax.nn.sigmoid(x)


_sc_mesh = plsc.VectorSubcoreMesh(core_axis_name="c", subcore_axis_name="s")


@functools.partial(
    pl.kernel,
    out_type=jax.ShapeDtypeStruct((NC, N_EDGES, HALF), jnp.float32),
    scratch_types=[
        pltpu.VMEM((NCHUNK, CHUNK), jnp.int32),
        pltpu.VMEM((CHUNK, HALF), jnp.float32),
        pltpu.SemaphoreType.DMA,
    ],
    mesh=_sc_mesh,
)
def _sc_gather(hcat_hbm, src2_hbm, out_hbm, idx_v, rows_v, sem):
    c = lax.axis_index("c")
    s = lax.axis_index("s")
    pltpu.sync_copy(src2_hbm.at[c, s], idx_v)

    def body(j, carry):
        base = s * EDGES_PER_SUB + j * CHUNK
        pltpu.async_copy(hcat_hbm.at[idx_v.at[j]], rows_v, sem).wait()
        pltpu.sync_copy(rows_v, out_hbm.at[c, pl.ds(base, CHUNK)])
        return carry

    lax.fori_loop(0, NCHUNK, body, 0)


@functools.partial(
    pl.kernel,
    out_type=jax.ShapeDtypeStruct((NC, N_PAD, HALF), jnp.float32),
    scratch_types=[
        pltpu.VMEM((NCHUNK, CHUNK), jnp.int32),
        pltpu.VMEM((CHUNK, HALF), jnp.float32),
        pltpu.VMEM_SHARED((N_PAD, HALF), jnp.float32),
        pltpu.SemaphoreType.DMA,
    ],
    mesh=_sc_mesh,
)
def _sc_scatter(msg_hbm, dst_hbm, zeros_hbm, out_hbm, idx_v, msg_v, aggr_sh, sem):
    c = lax.axis_index("c")
    s = lax.axis_index("s")
    pltpu.sync_copy(dst_hbm.at[s], idx_v)
    pltpu.sync_copy(zeros_hbm, aggr_sh.at[pl.ds(s * ROWS_PER_SUB, ROWS_PER_SUB)])
    plsc.subcore_barrier()

    def body(j, carry):
        base = s * EDGES_PER_SUB + j * CHUNK
        pltpu.sync_copy(msg_hbm.at[c, pl.ds(base, CHUNK)], msg_v)
        pltpu.sync_copy(msg_v, aggr_sh.at[idx_v.at[j]], add=True)
        return carry

    lax.fori_loop(0, NCHUNK, body, 0)
    plsc.subcore_barrier()
    pltpu.sync_copy(
        aggr_sh.at[pl.ds(s * ROWS_PER_SUB, ROWS_PER_SUB)],
        out_hbm.at[c, pl.ds(s * ROWS_PER_SUB, ROWS_PER_SUB)],
    )


BE = 3200


def _msg_body(rbf_ref, hsrc_ref, W1_ref, b1_ref, W2_ref, b2_ref, out_ref):
    g = _silu(jnp.dot(rbf_ref[...], W1_ref[...], preferred_element_type=jnp.float32)
              + b1_ref[...])
    gate = jax.nn.sigmoid(jnp.dot(g, W2_ref[...], preferred_element_type=jnp.float32)
                          + b2_ref[...])
    out_ref[0] = gate[:, :HALF] * hsrc_ref[0]
    out_ref[1] = gate[:, HALF:] * hsrc_ref[1]


def _msg_call(rbf, hsrc2, W1, b1, W2, b2):
    return pl.pallas_call(
        _msg_body,
        grid=(N_EDGES // BE,),
        in_specs=[
            pl.BlockSpec((BE, N_RBF), lambda i: (i, 0)),
            pl.BlockSpec((NC, BE, HALF), lambda i: (0, i, 0)),
            pl.BlockSpec((N_RBF, HIDDEN), lambda i: (0, 0)),
            pl.BlockSpec((1, HIDDEN), lambda i: (0, 0)),
            pl.BlockSpec((HIDDEN, HIDDEN), lambda i: (0, 0)),
            pl.BlockSpec((1, HIDDEN), lambda i: (0, 0)),
        ],
        out_specs=pl.BlockSpec((NC, BE, HALF), lambda i: (0, i, 0)),
        out_shape=jax.ShapeDtypeStruct((NC, N_EDGES, HALF), jnp.float32),
    )(rbf, hsrc2, W1, b1.reshape(1, HIDDEN), W2, b2.reshape(1, HIDDEN))


BN = 2000


def _upd_body(h_ref, aggr_ref, U1a_ref, U1b_ref, c1_ref, U2_ref, c2_ref, out_ref):
    h = h_ref[...]
    acc = jnp.dot(h, U1a_ref[...], preferred_element_type=jnp.float32)
    acc += jnp.dot(aggr_ref[0], U1b_ref[0], preferred_element_type=jnp.float32)
    acc += jnp.dot(aggr_ref[1], U1b_ref[1], preferred_element_type=jnp.float32)
    u = _silu(acc + c1_ref[...])
    out_ref[...] = h + jnp.dot(u, U2_ref[...], preferred_element_type=jnp.float32) \
        + c2_ref[...]


def _upd_call(h, aggr2, U1a, U1b, c1, U2, c2):
    return pl.pallas_call(
        _upd_body,
        grid=(N_NODES // BN,),
        in_specs=[
            pl.BlockSpec((BN, HIDDEN), lambda i: (i, 0)),
            pl.BlockSpec((NC, BN, HALF), lambda i: (0, i, 0)),
            pl.BlockSpec((HIDDEN, HIDDEN), lambda i: (0, 0)),
            pl.BlockSpec((NC, HALF, HIDDEN), lambda i: (0, 0, 0)),
            pl.BlockSpec((1, HIDDEN), lambda i: (0, 0)),
            pl.BlockSpec((HIDDEN, HIDDEN), lambda i: (0, 0)),
            pl.BlockSpec((1, HIDDEN), lambda i: (0, 0)),
        ],
        out_specs=pl.BlockSpec((BN, HIDDEN), lambda i: (i, 0)),
        out_shape=jax.ShapeDtypeStruct((N_NODES, HIDDEN), jnp.float32),
    )(h, aggr2, U1a, U1b, c1.reshape(1, HIDDEN), U2, c2.reshape(1, HIDDEN))


def kernel(h, edge_index, rbf, W1, b1, W2, b2, U1, c1, U2, c2):
    src = edge_index[0]
    dst = edge_index[1]
    hcat = h.reshape(N_NODES, NC, HALF).transpose(1, 0, 2).reshape(NC * N_NODES, HALF)
    src_r = src.reshape(NS, NCHUNK, CHUNK)
    src2 = jnp.stack([src_r, src_r + N_NODES])
    dst_r = dst.reshape(NS, NCHUNK, CHUNK)
    zeros = jnp.zeros((ROWS_PER_SUB, HALF), jnp.float32)

    hsrc2 = _sc_gather(hcat, src2)
    msg2 = _msg_call(rbf, hsrc2, W1, b1, W2, b2)
    aggr2 = _sc_scatter(msg2, dst_r, zeros)

    U1a = U1[:HIDDEN]
    U1b = U1[HIDDEN:].reshape(NC, HALF, HIDDEN)
    return _upd_call(h, aggr2, U1a, U1b, c1, U2, c2)


# speedup vs baseline: 2.8915x; 1.3339x over previous
import functools

import jax
import jax.numpy as jnp
from jax import lax
from jax.experimental import pallas as pl
from jax.experimental.pallas import tpu as pltpu
from jax.experimental.pallas import tpu_sc as plsc

N_NODES = 10000
N_EDGES = 160000
HIDDEN = 256
HALF = 128
N_RBF = 16

NC = 2
NS = 16
CHUNK = 80
EDGES_PER_SUB = N_EDGES // NS
NCHUNK = EDGES_PER_SUB // CHUNK
N_PAD = 10240
ROWS_PER_SUB = N_PAD // NS


def _silu(x):
    return x * jax.nn.sigmoid(x)


_sc_mesh = plsc.VectorSubcoreMesh(core_axis_name="c", subcore_axis_name="s")


NPAIR = (NCHUNK - 1) // 2


@functools.partial(
    pl.kernel,
    out_type=jax.ShapeDtypeStruct((NC, N_EDGES, HALF), jnp.float32),
    scratch_types=[
        pltpu.VMEM((NCHUNK, CHUNK), jnp.int32),
        pltpu.VMEM((CHUNK, HALF), jnp.float32),
        pltpu.VMEM((CHUNK, HALF), jnp.float32),
        pltpu.SemaphoreType.DMA,
        pltpu.SemaphoreType.DMA,
    ],
    mesh=_sc_mesh,
)
def _sc_gather(hcat_hbm, src2_hbm, out_hbm, idx_v, buf0, buf1, sem0, sem1):
    c = lax.axis_index("c")
    s = lax.axis_index("s")
    pltpu.sync_copy(src2_hbm.at[c, s], idx_v)
    ebase = s * EDGES_PER_SUB
    pltpu.async_copy(hcat_hbm.at[idx_v.at[0]], buf0, sem0)

    def body(t, carry):
        j0 = 2 * t
        pltpu.async_copy(hcat_hbm.at[idx_v.at[j0 + 1]], buf1, sem1)
        pltpu.make_async_copy(hcat_hbm.at[idx_v.at[j0]], buf0, sem0).wait()
        pltpu.sync_copy(buf0, out_hbm.at[c, pl.ds(ebase + j0 * CHUNK, CHUNK)])
        pltpu.async_copy(hcat_hbm.at[idx_v.at[j0 + 2]], buf0, sem0)
        pltpu.make_async_copy(hcat_hbm.at[idx_v.at[j0 + 1]], buf1, sem1).wait()
        pltpu.sync_copy(buf1, out_hbm.at[c, pl.ds(ebase + (j0 + 1) * CHUNK, CHUNK)])
        return carry

    lax.fori_loop(0, NPAIR, body, 0)
    j_last = NCHUNK - 1
    pltpu.make_async_copy(hcat_hbm.at[idx_v.at[j_last]], buf0, sem0).wait()
    pltpu.sync_copy(buf0, out_hbm.at[c, pl.ds(ebase + j_last * CHUNK, CHUNK)])


@functools.partial(
    pl.kernel,
    out_type=jax.ShapeDtypeStruct((NC, N_PAD, HALF), jnp.float32),
    scratch_types=[
        pltpu.VMEM((NCHUNK, CHUNK), jnp.int32),
        pltpu.VMEM((CHUNK, HALF), jnp.float32),
        pltpu.VMEM((CHUNK, HALF), jnp.float32),
        pltpu.VMEM_SHARED((N_PAD, HALF), jnp.float32),
        pltpu.SemaphoreType.DMA,
        pltpu.SemaphoreType.DMA,
    ],
    mesh=_sc_mesh,
)
def _sc_scatter(msg_hbm, dst_hbm, zeros_hbm, out_hbm, idx_v, buf0, buf1, aggr_sh,
                sem0, sem1):
    c = lax.axis_index("c")
    s = lax.axis_index("s")
    pltpu.sync_copy(dst_hbm.at[s], idx_v)
    pltpu.sync_copy(zeros_hbm, aggr_sh.at[pl.ds(s * ROWS_PER_SUB, ROWS_PER_SUB)])
    plsc.subcore_barrier()
    ebase = s * EDGES_PER_SUB
    pltpu.async_copy(msg_hbm.at[c, pl.ds(ebase, CHUNK)], buf0, sem0)

    def body(t, carry):
        j0 = 2 * t
        pltpu.async_copy(msg_hbm.at[c, pl.ds(ebase + (j0 + 1) * CHUNK, CHUNK)],
                         buf1, sem1)
        pltpu.make_async_copy(msg_hbm.at[c, pl.ds(ebase + j0 * CHUNK, CHUNK)],
                              buf0, sem0).wait()
        pltpu.sync_copy(buf0, aggr_sh.at[idx_v.at[j0]], add=True)
        pltpu.async_copy(msg_hbm.at[c, pl.ds(ebase + (j0 + 2) * CHUNK, CHUNK)],
                         buf0, sem0)
        pltpu.make_async_copy(msg_hbm.at[c, pl.ds(ebase + (j0 + 1) * CHUNK, CHUNK)],
                              buf1, sem1).wait()
        pltpu.sync_copy(buf1, aggr_sh.at[idx_v.at[j0 + 1]], add=True)
        return carry

    lax.fori_loop(0, NPAIR, body, 0)
    j_last = NCHUNK - 1
    pltpu.make_async_copy(msg_hbm.at[c, pl.ds(ebase + j_last * CHUNK, CHUNK)],
                          buf0, sem0).wait()
    pltpu.sync_copy(buf0, aggr_sh.at[idx_v.at[j_last]], add=True)
    plsc.subcore_barrier()
    pltpu.sync_copy(
        aggr_sh.at[pl.ds(s * ROWS_PER_SUB, ROWS_PER_SUB)],
        out_hbm.at[c, pl.ds(s * ROWS_PER_SUB, ROWS_PER_SUB)],
    )


BE = 3200


def _msg_body(rbf_ref, hsrc_ref, W1_ref, b1_ref, W2_ref, b2_ref, out_ref):
    g = _silu(jnp.dot(rbf_ref[...], W1_ref[...], preferred_element_type=jnp.float32)
              + b1_ref[...])
    gate = jax.nn.sigmoid(jnp.dot(g, W2_ref[...], preferred_element_type=jnp.float32)
                          + b2_ref[...])
    out_ref[0] = gate[:, :HALF] * hsrc_ref[0]
    out_ref[1] = gate[:, HALF:] * hsrc_ref[1]


def _msg_call(rbf, hsrc2, W1, b1, W2, b2):
    return pl.pallas_call(
        _msg_body,
        grid=(N_EDGES // BE,),
        in_specs=[
            pl.BlockSpec((BE, N_RBF), lambda i: (i, 0)),
            pl.BlockSpec((NC, BE, HALF), lambda i: (0, i, 0)),
            pl.BlockSpec((N_RBF, HIDDEN), lambda i: (0, 0)),
            pl.BlockSpec((1, HIDDEN), lambda i: (0, 0)),
            pl.BlockSpec((HIDDEN, HIDDEN), lambda i: (0, 0)),
            pl.BlockSpec((1, HIDDEN), lambda i: (0, 0)),
        ],
        out_specs=pl.BlockSpec((NC, BE, HALF), lambda i: (0, i, 0)),
        out_shape=jax.ShapeDtypeStruct((NC, N_EDGES, HALF), jnp.float32),
    )(rbf, hsrc2, W1, b1.reshape(1, HIDDEN), W2, b2.reshape(1, HIDDEN))


BN = 2000


def _upd_body(h_ref, aggr_ref, U1a_ref, U1b_ref, c1_ref, U2_ref, c2_ref, out_ref):
    h = h_ref[...]
    acc = jnp.dot(h, U1a_ref[...], preferred_element_type=jnp.float32)
    acc += jnp.dot(aggr_ref[0], U1b_ref[0], preferred_element_type=jnp.float32)
    acc += jnp.dot(aggr_ref[1], U1b_ref[1], preferred_element_type=jnp.float32)
    u = _silu(acc + c1_ref[...])
    out_ref[...] = h + jnp.dot(u, U2_ref[...], preferred_element_type=jnp.float32) \
        + c2_ref[...]


def _upd_call(h, aggr2, U1a, U1b, c1, U2, c2):
    return pl.pallas_call(
        _upd_body,
        grid=(N_NODES // BN,),
        in_specs=[
            pl.BlockSpec((BN, HIDDEN), lambda i: (i, 0)),
            pl.BlockSpec((NC, BN, HALF), lambda i: (0, i, 0)),
            pl.BlockSpec((HIDDEN, HIDDEN), lambda i: (0, 0)),
            pl.BlockSpec((NC, HALF, HIDDEN), lambda i: (0, 0, 0)),
            pl.BlockSpec((1, HIDDEN), lambda i: (0, 0)),
            pl.BlockSpec((HIDDEN, HIDDEN), lambda i: (0, 0)),
            pl.BlockSpec((1, HIDDEN), lambda i: (0, 0)),
        ],
        out_specs=pl.BlockSpec((BN, HIDDEN), lambda i: (i, 0)),
        out_shape=jax.ShapeDtypeStruct((N_NODES, HIDDEN), jnp.float32),
    )(h, aggr2, U1a, U1b, c1.reshape(1, HIDDEN), U2, c2.reshape(1, HIDDEN))


def kernel(h, edge_index, rbf, W1, b1, W2, b2, U1, c1, U2, c2):
    src = edge_index[0]
    dst = edge_index[1]
    hcat = h.reshape(N_NODES, NC, HALF).transpose(1, 0, 2).reshape(NC * N_NODES, HALF)
    src_r = src.reshape(NS, NCHUNK, CHUNK)
    src2 = jnp.stack([src_r, src_r + N_NODES])
    dst_r = dst.reshape(NS, NCHUNK, CHUNK)
    zeros = jnp.zeros((ROWS_PER_SUB, HALF), jnp.float32)

    hsrc2 = _sc_gather(hcat, src2)
    msg2 = _msg_call(rbf, hsrc2, W1, b1, W2, b2)
    aggr2 = _sc_scatter(msg2, dst_r, zeros)

    U1a = U1[:HIDDEN]
    U1b = U1[HIDDEN:].reshape(NC, HALF, HIDDEN)
    return _upd_call(h, aggr2, U1a, U1b, c1, U2, c2)
